# Initial kernel scaffold; baseline (speedup 1.0000x reference)
import jax
import jax.numpy as jnp
from jax.experimental import pallas as pl


def kernel(x, gate_w, w1, w2, b1, b2):
    raise NotImplementedError("write your pallas kernel here")


# speedup vs baseline: 37.0777x; 37.0777x over previous
import math

import jax
import jax.numpy as jnp
from jax.experimental import pallas as pl
from jax.experimental.pallas import tpu as pltpu

_E, _TOPK = 8, 2


def _moe_kernel(x_ref, gw_ref, w1_ref, w2_ref, b1_ref, b2_ref,
                out_ref, aux_ref, coeff_ref):
    e = pl.program_id(0)
    x = x_ref[...]

    @pl.when(e == 0)
    def _routing():
        logits = jnp.dot(x, gw_ref[...].T, preferred_element_type=jnp.float32)
        cols = jax.lax.broadcasted_iota(jnp.int32, logits.shape, 1)
        m1 = jnp.max(logits, axis=1, keepdims=True)
        idx1 = jnp.min(jnp.where(logits == m1, cols, _E), axis=1, keepdims=True)
        is1 = cols == idx1
        logits2 = jnp.where(is1, -jnp.inf, logits)
        m2 = jnp.max(logits2, axis=1, keepdims=True)
        idx2 = jnp.min(jnp.where(logits2 == m2, cols, _E), axis=1, keepdims=True)
        is2 = cols == idx2
        ed = jnp.exp(m2 - m1)
        denom = 1.0 + ed
        coeff = jnp.where(is1, 1.0 / denom, 0.0) + jnp.where(is2, ed / denom, 0.0)
        coeff_ref[...] = coeff
        p = jnp.exp(logits - m1)
        probs = p / jnp.sum(p, axis=1, keepdims=True)
        frac_probs = jnp.mean(probs, axis=0)
        frac_tokens = jnp.mean(is1.astype(jnp.float32), axis=0)
        aux = (_E * jnp.sum(frac_tokens * frac_probs)
               + jnp.mean(logits * logits) * 0.001)
        aux_ref[...] = jnp.broadcast_to(aux, aux_ref.shape)
        out_ref[...] = jnp.zeros_like(out_ref)

    h = jnp.dot(x, w1_ref[0], preferred_element_type=jnp.float32) + b1_ref[0]
    h = 0.5 * h * (1.0 + jax.lax.erf(h * (1.0 / math.sqrt(2.0))))
    y = jnp.dot(h, w2_ref[0], preferred_element_type=jnp.float32) + b2_ref[0]
    cols = jax.lax.broadcasted_iota(jnp.int32, coeff_ref.shape, 1)
    ce = jnp.sum(jnp.where(cols == e, coeff_ref[...], 0.0), axis=1, keepdims=True)
    out_ref[...] += ce * y


def kernel(x, gate_w, w1, w2, b1, b2):
    B, T, D = x.shape
    E, _, F = w1.shape
    N = B * T
    x2 = x.reshape(N, D)
    out, aux = pl.pallas_call(
        _moe_kernel,
        grid=(E,),
        in_specs=[
            pl.BlockSpec((N, D), lambda e: (0, 0)),
            pl.BlockSpec((E, D), lambda e: (0, 0)),
            pl.BlockSpec((1, D, F), lambda e: (e, 0, 0)),
            pl.BlockSpec((1, F, D), lambda e: (e, 0, 0)),
            pl.BlockSpec((1, 1, F), lambda e: (e, 0, 0)),
            pl.BlockSpec((1, 1, D), lambda e: (e, 0, 0)),
        ],
        out_specs=[
            pl.BlockSpec((N, D), lambda e: (0, 0)),
            pl.BlockSpec((1, 1), lambda e: (0, 0)),
        ],
        out_shape=[
            jax.ShapeDtypeStruct((N, D), jnp.float32),
            jax.ShapeDtypeStruct((1, 1), jnp.float32),
        ],
        scratch_shapes=[pltpu.VMEM((N, E), jnp.float32)],
    )(x2, gate_w, w1, w2, b1.reshape(E, 1, F), b2.reshape(E, 1, D))
    return out.reshape(B, T, D), aux[0, 0]
